# baseline (speedup 1.0000x reference)
import functools

import jax
import jax.numpy as jnp
from jax import lax
from jax.experimental import pallas as pl
from jax.experimental.pallas import tpu as pltpu
from jax.experimental.pallas import tpu_sc as plsc

N = 10000
E = 320000
D_IN = 128
H = 256
HH = 128
NOBJ = 1598
NOBJ_PAD = 1600
NDEV = 7
NDEV_PAD = 8
EPS = 1e-5

NSUB = 16
ES = E // NSUB
K = 80
CPB = 10
NIB = 25
NPAD = 10240
RPS = NPAD // NSUB

RB = 1000
NRB = N // RB

_f32 = jnp.float32


def _dotT(a, b):
    return lax.dot_general(a, b, (((1,), (1,)), ((), ())),
                           preferred_element_type=_f32)


def _make_agg():
    mesh = plsc.VectorSubcoreMesh(core_axis_name="c", subcore_axis_name="s")
    NBUF = 4
    scratch = (
        [pltpu.VMEM((CPB, K), jnp.int32)] * 2 +
        [pltpu.VMEM((K, HH), _f32)] * NBUF +
        [pltpu.SemaphoreType.DMA] * (2 * NBUF) +
        [pltpu.VMEM_SHARED((NPAD, HH), _f32)]
    )

    def body(h2, ei4, zacc, agg_out, src_loc, dst_loc, *rest):
        rows = rest[:NBUF]
        semg = rest[NBUF:2 * NBUF]
        sems = rest[2 * NBUF:3 * NBUF]
        acc = rest[3 * NBUF]
        c = lax.axis_index("c")
        s = lax.axis_index("s")

        pltpu.sync_copy(zacc, acc.at[pl.ds(s * RPS, RPS)])
        plsc.subcore_barrier()
        off = c * N

        def iblock(ib, _):
            pltpu.sync_copy(ei4.at[0, s * NIB + ib], src_loc)
            pltpu.sync_copy(ei4.at[1, s * NIB + ib], dst_loc)

            def fix(i, _):
                r = i // (K // 16)
                q = (i % (K // 16)) * 16
                src_loc[r, pl.ds(q, 16)] = src_loc[r, pl.ds(q, 16)] + off
                return 0

            lax.fori_loop(0, CPB * (K // 16), fix, 0)

            gd = [None] * CPB
            sd = [None] * CPB
            for r in range(CPB):
                if r >= NBUF:
                    sd[r - NBUF].wait()
                gd[r] = pltpu.async_copy(h2.at[src_loc.at[r]],
                                         rows[r % NBUF], semg[r % NBUF])
                if r >= 1:
                    gd[r - 1].wait()
                    sd[r - 1] = pltpu.async_copy(
                        rows[(r - 1) % NBUF], acc.at[dst_loc.at[r - 1]],
                        sems[(r - 1) % NBUF], add=True)
            gd[CPB - 1].wait()
            sd[CPB - 1] = pltpu.async_copy(
                rows[(CPB - 1) % NBUF], acc.at[dst_loc.at[CPB - 1]],
                sems[(CPB - 1) % NBUF], add=True)
            for r in range(CPB - NBUF, CPB):
                sd[r].wait()
            return 0

        lax.fori_loop(0, NIB, iblock, 0)
        plsc.subcore_barrier()

        pltpu.sync_copy(acc.at[pl.ds(s * RPS, RPS)],
                        agg_out.at[c].at[pl.ds(s * RPS, RPS)])

    return pl.kernel(body,
                     out_type=jax.ShapeDtypeStruct((2, NPAD, HH), _f32),
                     mesh=mesh, scratch_types=scratch)


_agg = _make_agg()

def _make_deg():
    mesh = plsc.VectorSubcoreMesh(core_axis_name="c", subcore_axis_name="s")
    scratch = [
        pltpu.VMEM((CPB, K), jnp.int32),
        pltpu.VMEM((K, HH), _f32),
        pltpu.VMEM_SHARED((NPAD, HH), _f32),
    ]

    def body(ei4, zacc, onesI, deg_out, dst_loc, ones_buf, acc):
        c = lax.axis_index("c")
        s = lax.axis_index("s")
        pltpu.sync_copy(zacc, acc.at[pl.ds(s * RPS, RPS)])
        pltpu.sync_copy(onesI, ones_buf)
        plsc.subcore_barrier()
        base = c * (CPB // 2)

        def iblock(ib, _):
            pltpu.sync_copy(ei4.at[1, s * NIB + ib], dst_loc)

            def chunk(r, _):
                pltpu.sync_copy(ones_buf, acc.at[dst_loc.at[base + r]],
                                add=True)
                return 0

            lax.fori_loop(0, CPB // 2, chunk, 0)
            return 0

        lax.fori_loop(0, NIB, iblock, 0)
        plsc.subcore_barrier()
        pltpu.sync_copy(acc.at[pl.ds(s * RPS, RPS)],
                        deg_out.at[c].at[pl.ds(s * RPS, RPS)])

    return pl.kernel(body,
                     out_type=jax.ShapeDtypeStruct((2, NPAD, HH), _f32),
                     mesh=mesh, scratch_types=scratch)


_deg_call = _make_deg()


def _input_body(x_ref, W_ref, b_ref, g_ref, bb_ref, out_ref,
                t_sc, sum_sc, sq_sc):
    p = pl.program_id(0)
    j = pl.program_id(1)

    @pl.when(p == 0)
    def _():
        t = _dotT(x_ref[...], W_ref[...]) + b_ref[...]
        t_sc[pl.ds(j * RB, RB), :] = t
        s_blk = jnp.sum(t, axis=0, keepdims=True)
        q_blk = jnp.sum(t * t, axis=0, keepdims=True)

        @pl.when(j == 0)
        def _():
            sum_sc[...] = s_blk
            sq_sc[...] = q_blk

        @pl.when(j > 0)
        def _():
            sum_sc[...] += s_blk
            sq_sc[...] += q_blk

    @pl.when(p == 1)
    def _():
        t = t_sc[pl.ds(j * RB, RB), :]
        mu = sum_sc[...] / N
        var = sq_sc[...] / N - mu * mu
        hn = jnp.maximum(
            (t - mu) / jnp.sqrt(var + EPS) * g_ref[...] + bb_ref[...], 0.0)
        out_ref[0] = hn[:, :HH]
        out_ref[1] = hn[:, HH:]


_input_call = pl.pallas_call(
    _input_body,
    grid=(2, NRB),
    in_specs=[
        pl.BlockSpec((RB, D_IN),
                     lambda p, j: (jnp.where(p == 0, j, NRB - 1), 0)),
        pl.BlockSpec((H, D_IN), lambda p, j: (0, 0)),
        pl.BlockSpec((1, H), lambda p, j: (0, 0)),
        pl.BlockSpec((1, H), lambda p, j: (0, 0)),
        pl.BlockSpec((1, H), lambda p, j: (0, 0)),
    ],
    out_specs=[
        pl.BlockSpec((2, RB, HH), lambda p, j: (0, j, 0)),
    ],
    out_shape=[jax.ShapeDtypeStruct((2, N, HH), _f32)],
    scratch_shapes=[pltpu.VMEM((N, H), _f32),
                    pltpu.VMEM((1, H), _f32),
                    pltpu.VMEM((1, H), _f32)],
)


def _make_pre(l):
    def body(hL_ref, hR_ref, Wr_ref, bl_ref, out_ref):
        h = jnp.concatenate([hL_ref[0], hR_ref[0]], axis=1)
        r = _dotT(h, Wr_ref[...]) + bl_ref[...]
        if l > 0:
            r = r + h
        out_ref[...] = r

    return pl.pallas_call(
        body,
        grid=(NRB,),
        in_specs=[
            pl.BlockSpec((1, RB, HH), lambda j: (0, j, 0)),
            pl.BlockSpec((1, RB, HH), lambda j: (1, j, 0)),
            pl.BlockSpec((H, H), lambda j: (0, 0)),
            pl.BlockSpec((1, H), lambda j: (0, 0)),
        ],
        out_specs=[pl.BlockSpec((RB, H), lambda j: (j, 0))],
        out_shape=[jax.ShapeDtypeStruct((N, H), _f32)],
    )


_pre0 = _make_pre(0)
_pre1 = _make_pre(1)
_pre2 = _make_pre(2)


def _make_layer(l):
    split_out = l < 2

    def body(aggL_ref, aggR_ref, R_ref, *rest):
        if l == 0:
            dh0_ref, dh1_ref, Wl_ref, g_ref, b_ref = rest[:5]
            hout_ref, deg_out = rest[5:7]
            t_sc, sum_sc, sq_sc = rest[7:]
        else:
            dh_ref, Wl_ref, g_ref, b_ref = rest[:4]
            hout_ref = rest[4]
            t_sc, sum_sc, sq_sc = rest[5:]
        p = pl.program_id(0)
        j = pl.program_id(1)
        if l == 0:
            deg = jnp.maximum(dh0_ref[0][:, :1] + dh1_ref[0][:, :1], 1.0)
        else:
            deg = dh_ref[...]

        @pl.when(p == 0)
        def _():
            agg = jnp.concatenate([aggL_ref[0], aggR_ref[0]], axis=1) / deg
            t = _dotT(agg, Wl_ref[...]) + R_ref[...]
            t_sc[pl.ds(j * RB, RB), :] = t
            s_blk = jnp.sum(t, axis=0, keepdims=True)
            q_blk = jnp.sum(t * t, axis=0, keepdims=True)

            @pl.when(j == 0)
            def _():
                sum_sc[...] = s_blk
                sq_sc[...] = q_blk

            @pl.when(j > 0)
            def _():
                sum_sc[...] += s_blk
                sq_sc[...] += q_blk

        if l == 0:
            deg_out[...] = deg

        @pl.when(p == 1)
        def _():
            t = t_sc[pl.ds(j * RB, RB), :]
            mu = sum_sc[...] / N
            var = sq_sc[...] / N - mu * mu
            hn = jnp.maximum(
                (t - mu) / jnp.sqrt(var + EPS) * g_ref[...] + b_ref[...], 0.0)
            if split_out:
                hout_ref[0] = hn[:, :HH]
                hout_ref[1] = hn[:, HH:]
            else:
                hout_ref[...] = hn

    def _pj(j_of):
        return lambda p, j: j_of(jnp.where(p == 0, j, NRB - 1))

    if l == 0:
        dh_specs = [pl.BlockSpec((1, RB, HH), _pj(lambda j: (0, j, 0))),
                    pl.BlockSpec((1, RB, HH), _pj(lambda j: (1, j, 0)))]
    else:
        dh_specs = [pl.BlockSpec((RB, 1), _pj(lambda j: (j, 0)))]
    in_specs = [
        pl.BlockSpec((1, RB, HH), _pj(lambda j: (0, j, 0))),
        pl.BlockSpec((1, RB, HH), _pj(lambda j: (1, j, 0))),
        pl.BlockSpec((RB, H), _pj(lambda j: (j, 0))),
    ] + dh_specs + [
        pl.BlockSpec((H, H), lambda p, j: (0, 0)),
        pl.BlockSpec((1, H), lambda p, j: (0, 0)),
        pl.BlockSpec((1, H), lambda p, j: (0, 0)),
    ]
    if split_out:
        out_specs = [pl.BlockSpec((2, RB, HH), lambda p, j: (0, j, 0))]
        out_shape = [jax.ShapeDtypeStruct((2, N, HH), _f32)]
    else:
        out_specs = [pl.BlockSpec((RB, H), lambda p, j: (j, 0))]
        out_shape = [jax.ShapeDtypeStruct((N, H), _f32)]
    if l == 0:
        out_specs.append(pl.BlockSpec(
            (RB, 1), lambda p, j: (jnp.where(p == 0, j, NRB - 1), 0)))
        out_shape.append(jax.ShapeDtypeStruct((N, 1), _f32))
    return pl.pallas_call(
        body,
        grid=(2, NRB),
        in_specs=in_specs,
        out_specs=out_specs,
        out_shape=out_shape,
        scratch_shapes=[pltpu.VMEM((N, H), _f32),
                        pltpu.VMEM((1, H), _f32),
                        pltpu.VMEM((1, H), _f32)],
    )


_layer0 = _make_layer(0)
_layer1 = _make_layer(1)
_layer2 = _make_layer(2)


def _heads_body(h_ref, We_ref, be_ref, Wo1_ref, bo1_ref, Wo2_ref, bo2_ref,
                Wd1_ref, bd1_ref, Wd2_ref, bd2_ref,
                emb_ref, obj_ref, dev_ref, sum_sc, max_sc):
    i = pl.program_id(0)
    h = h_ref[...]
    emb_ref[...] = _dotT(h, We_ref[...]) + be_ref[...]
    t1 = jnp.maximum(_dotT(h, Wo1_ref[...]) + bo1_ref[...], 0.0)
    obj_ref[...] = (_dotT(t1, Wo2_ref[...]) + bo2_ref[...])[:, :NOBJ]
    s_blk = jnp.sum(h, axis=0, keepdims=True)
    m_blk = jnp.max(h, axis=0, keepdims=True)

    @pl.when(i == 0)
    def _():
        sum_sc[...] = s_blk
        max_sc[...] = m_blk

    @pl.when(i > 0)
    def _():
        sum_sc[...] += s_blk
        max_sc[...] = jnp.maximum(max_sc[...], m_blk)

    grepr = jnp.concatenate([sum_sc[...] / N, max_sc[...]], axis=1)
    d1 = jnp.maximum(_dotT(grepr, Wd1_ref[...]) + bd1_ref[...], 0.0)
    dev_ref[...] = (_dotT(d1, Wd2_ref[...]) + bd2_ref[...])[:, :NDEV]


_heads_call = pl.pallas_call(
    _heads_body,
    grid=(NRB,),
    in_specs=[
        pl.BlockSpec((RB, H), lambda i: (i, 0)),
        pl.BlockSpec((H, H), lambda i: (0, 0)),
        pl.BlockSpec((1, H), lambda i: (0, 0)),
        pl.BlockSpec((H, H), lambda i: (0, 0)),
        pl.BlockSpec((1, H), lambda i: (0, 0)),
        pl.BlockSpec((NOBJ_PAD, H), lambda i: (0, 0)),
        pl.BlockSpec((1, NOBJ_PAD), lambda i: (0, 0)),
        pl.BlockSpec((H, 2 * H), lambda i: (0, 0)),
        pl.BlockSpec((1, H), lambda i: (0, 0)),
        pl.BlockSpec((NDEV_PAD, H), lambda i: (0, 0)),
        pl.BlockSpec((1, NDEV_PAD), lambda i: (0, 0)),
    ],
    out_specs=[
        pl.BlockSpec((RB, H), lambda i: (i, 0)),
        pl.BlockSpec((RB, NOBJ), lambda i: (i, 0)),
        pl.BlockSpec((1, NDEV), lambda i: (0, 0)),
    ],
    out_shape=[jax.ShapeDtypeStruct((N, H), _f32),
               jax.ShapeDtypeStruct((N, NOBJ), _f32),
               jax.ShapeDtypeStruct((1, NDEV), _f32)],
    scratch_shapes=[pltpu.VMEM((1, H), _f32),
                    pltpu.VMEM((1, H), _f32)],
)


def kernel(x, edge_index, edge_attr,
           W_in, b_in, bn_in_g, bn_in_b,
           W_emb, b_emb,
           W_o1, b_o1, W_o2, b_o2,
           W_d1, b_d1, W_d2, b_d2,
           sage0_Wl, sage0_bl, sage0_Wr, norm0_g, norm0_b,
           sage1_Wl, sage1_bl, sage1_Wr, norm1_g, norm1_b,
           sage2_Wl, sage2_bl, sage2_Wr, norm2_g, norm2_b):
    row = lambda v: v.reshape(1, -1)
    ei4 = edge_index.reshape(2, NSUB * NIB, CPB, K)
    zacc = jnp.zeros((RPS, HH), _f32)
    onesI = jnp.ones((K, HH), _f32)

    (h0,) = _input_call(x, W_in, row(b_in), row(bn_in_g), row(bn_in_b))
    deg128 = _deg_call(ei4, zacc, onesI)
    a0 = _agg(h0.reshape(2 * N, HH), ei4, zacc)
    (R0,) = _pre0(h0, h0, sage0_Wr, row(sage0_bl))
    h1, deg = _layer0(a0, a0, R0, deg128, deg128,
                      sage0_Wl, row(norm0_g), row(norm0_b))
    a1 = _agg(h1.reshape(2 * N, HH), ei4, zacc)
    (R1,) = _pre1(h1, h1, sage1_Wr, row(sage1_bl))
    (h2,) = _layer1(a1, a1, R1, deg,
                    sage1_Wl, row(norm1_g), row(norm1_b))
    a2 = _agg(h2.reshape(2 * N, HH), ei4, zacc)
    (R2,) = _pre2(h2, h2, sage2_Wr, row(sage2_bl))
    (h3,) = _layer2(a2, a2, R2, deg,
                    sage2_Wl, row(norm2_g), norm2_b.reshape(1, -1))

    W_o2p = jnp.pad(W_o2, ((0, NOBJ_PAD - NOBJ), (0, 0)))
    b_o2p = jnp.pad(b_o2, (0, NOBJ_PAD - NOBJ)).reshape(1, -1)
    W_d2p = jnp.pad(W_d2, ((0, NDEV_PAD - NDEV), (0, 0)))
    b_d2p = jnp.pad(b_d2, (0, NDEV_PAD - NDEV)).reshape(1, -1)
    emb, obj, dev = _heads_call(h3, W_emb, row(b_emb),
                                W_o1, row(b_o1), W_o2p, b_o2p,
                                W_d1, row(b_d1), W_d2p, b_d2p)
    return emb, obj, dev, h3


# speedup vs baseline: 1.0108x; 1.0108x over previous
import functools

import jax
import jax.numpy as jnp
from jax import lax
from jax.experimental import pallas as pl
from jax.experimental.pallas import tpu as pltpu
from jax.experimental.pallas import tpu_sc as plsc

N = 10000
E = 320000
D_IN = 128
H = 256
HH = 128
NOBJ = 1598
NOBJ_PAD = 1600
NDEV = 7
NDEV_PAD = 8
EPS = 1e-5

NSUB = 16
ES = E // NSUB
K = 80
CPB = 10
NIB = 25
NPAD = 10240
RPS = NPAD // NSUB

RB = 1000
NRB = N // RB

_f32 = jnp.float32


def _dotT(a, b):
    return lax.dot_general(a, b, (((1,), (1,)), ((), ())),
                           preferred_element_type=_f32)


def _make_agg():
    mesh = plsc.VectorSubcoreMesh(core_axis_name="c", subcore_axis_name="s")
    NBUF = 4
    scratch = (
        [pltpu.VMEM((CPB, K), jnp.int32)] * 2 +
        [pltpu.VMEM((K, HH), _f32)] * NBUF +
        [pltpu.SemaphoreType.DMA] * (2 * NBUF) +
        [pltpu.VMEM_SHARED((NPAD, HH), _f32)]
    )

    def body(h2, ei4, zacc, agg_out, src_loc, dst_loc, *rest):
        rows = rest[:NBUF]
        semg = rest[NBUF:2 * NBUF]
        sems = rest[2 * NBUF:3 * NBUF]
        acc = rest[3 * NBUF]
        c = lax.axis_index("c")
        s = lax.axis_index("s")

        pltpu.sync_copy(zacc, acc.at[pl.ds(s * RPS, RPS)])
        plsc.subcore_barrier()
        off = c * N

        def iblock(ib, _):
            pltpu.sync_copy(ei4.at[0, s * NIB + ib], src_loc)
            pltpu.sync_copy(ei4.at[1, s * NIB + ib], dst_loc)

            def fix(i, _):
                r = i // (K // 16)
                q = (i % (K // 16)) * 16
                src_loc[r, pl.ds(q, 16)] = src_loc[r, pl.ds(q, 16)] + off
                return 0

            lax.fori_loop(0, CPB * (K // 16), fix, 0)

            gd = [None] * CPB
            sd = [None] * CPB
            for r in range(CPB):
                if r >= NBUF:
                    sd[r - NBUF].wait()
                gd[r] = pltpu.async_copy(h2.at[src_loc.at[r]],
                                         rows[r % NBUF], semg[r % NBUF])
                if r >= 1:
                    gd[r - 1].wait()
                    sd[r - 1] = pltpu.async_copy(
                        rows[(r - 1) % NBUF], acc.at[dst_loc.at[r - 1]],
                        sems[(r - 1) % NBUF], add=True)
            gd[CPB - 1].wait()
            sd[CPB - 1] = pltpu.async_copy(
                rows[(CPB - 1) % NBUF], acc.at[dst_loc.at[CPB - 1]],
                sems[(CPB - 1) % NBUF], add=True)
            for r in range(CPB - NBUF, CPB):
                sd[r].wait()
            return 0

        lax.fori_loop(0, NIB, iblock, 0)
        plsc.subcore_barrier()

        pltpu.sync_copy(acc.at[pl.ds(s * RPS, RPS)],
                        agg_out.at[c].at[pl.ds(s * RPS, RPS)])

    return pl.kernel(body,
                     out_type=jax.ShapeDtypeStruct((2, NPAD, HH), _f32),
                     mesh=mesh, scratch_types=scratch)


_agg = _make_agg()

def _make_deg():
    mesh = plsc.VectorSubcoreMesh(core_axis_name="c", subcore_axis_name="s")
    scratch = [
        pltpu.VMEM((CPB, K), jnp.int32),
        pltpu.VMEM((K, HH), _f32),
        pltpu.VMEM_SHARED((NPAD, HH), _f32),
    ]

    def body(ei4, zacc, onesI, deg_out, dst_loc, ones_buf, acc):
        c = lax.axis_index("c")
        s = lax.axis_index("s")
        pltpu.sync_copy(zacc, acc.at[pl.ds(s * RPS, RPS)])
        pltpu.sync_copy(onesI, ones_buf)
        plsc.subcore_barrier()
        base = c * (CPB // 2)

        def iblock(ib, _):
            pltpu.sync_copy(ei4.at[1, s * NIB + ib], dst_loc)

            def chunk(r, _):
                pltpu.sync_copy(ones_buf, acc.at[dst_loc.at[base + r]],
                                add=True)
                return 0

            lax.fori_loop(0, CPB // 2, chunk, 0)
            return 0

        lax.fori_loop(0, NIB, iblock, 0)
        plsc.subcore_barrier()
        pltpu.sync_copy(acc.at[pl.ds(s * RPS, RPS)],
                        deg_out.at[c].at[pl.ds(s * RPS, RPS)])

    return pl.kernel(body,
                     out_type=jax.ShapeDtypeStruct((2, NPAD, HH), _f32),
                     mesh=mesh, scratch_types=scratch)


_deg_call = _make_deg()


def _input_body(x_ref, W_ref, b_ref, g_ref, bb_ref, out_ref,
                t_sc, sum_sc, sq_sc):
    p = pl.program_id(0)
    j = pl.program_id(1)

    @pl.when(p == 0)
    def _():
        t = _dotT(x_ref[...], W_ref[...]) + b_ref[...]
        t_sc[pl.ds(j * RB, RB), :] = t
        s_blk = jnp.sum(t, axis=0, keepdims=True)
        q_blk = jnp.sum(t * t, axis=0, keepdims=True)

        @pl.when(j == 0)
        def _():
            sum_sc[...] = s_blk
            sq_sc[...] = q_blk

        @pl.when(j > 0)
        def _():
            sum_sc[...] += s_blk
            sq_sc[...] += q_blk

    @pl.when(p == 1)
    def _():
        t = t_sc[pl.ds(j * RB, RB), :]
        mu = sum_sc[...] / N
        var = sq_sc[...] / N - mu * mu
        hn = jnp.maximum(
            (t - mu) / jnp.sqrt(var + EPS) * g_ref[...] + bb_ref[...], 0.0)
        out_ref[0] = hn[:, :HH]
        out_ref[1] = hn[:, HH:]


_input_call = pl.pallas_call(
    _input_body,
    grid=(2, NRB),
    in_specs=[
        pl.BlockSpec((RB, D_IN),
                     lambda p, j: (jnp.where(p == 0, j, NRB - 1), 0)),
        pl.BlockSpec((H, D_IN), lambda p, j: (0, 0)),
        pl.BlockSpec((1, H), lambda p, j: (0, 0)),
        pl.BlockSpec((1, H), lambda p, j: (0, 0)),
        pl.BlockSpec((1, H), lambda p, j: (0, 0)),
    ],
    out_specs=[
        pl.BlockSpec((2, RB, HH), lambda p, j: (0, j, 0)),
    ],
    out_shape=[jax.ShapeDtypeStruct((2, N, HH), _f32)],
    scratch_shapes=[pltpu.VMEM((N, H), _f32),
                    pltpu.VMEM((1, H), _f32),
                    pltpu.VMEM((1, H), _f32)],
)


def _make_layer(l):
    split_out = l < 2

    def body(aggL_ref, aggR_ref, hL_ref, hR_ref, *rest):
        if l == 0:
            dh0_ref, dh1_ref, Wl_ref, bl_ref, Wr_ref, g_ref, b_ref = rest[:7]
            hout_ref, deg_out = rest[7:9]
            t_sc, sum_sc, sq_sc = rest[9:]
        else:
            dh_ref, Wl_ref, bl_ref, Wr_ref, g_ref, b_ref = rest[:6]
            hout_ref = rest[6]
            t_sc, sum_sc, sq_sc = rest[7:]
        p = pl.program_id(0)
        j = pl.program_id(1)
        if l == 0:
            deg = jnp.maximum(dh0_ref[0][:, :1] + dh1_ref[0][:, :1], 1.0)
        else:
            deg = dh_ref[...]

        @pl.when(p == 0)
        def _():
            agg = jnp.concatenate([aggL_ref[0], aggR_ref[0]], axis=1) / deg
            h = jnp.concatenate([hL_ref[0], hR_ref[0]], axis=1)
            t = _dotT(agg, Wl_ref[...]) + bl_ref[...] + _dotT(h, Wr_ref[...])
            if l > 0:
                t = t + h
            t_sc[pl.ds(j * RB, RB), :] = t
            s_blk = jnp.sum(t, axis=0, keepdims=True)
            q_blk = jnp.sum(t * t, axis=0, keepdims=True)

            @pl.when(j == 0)
            def _():
                sum_sc[...] = s_blk
                sq_sc[...] = q_blk

            @pl.when(j > 0)
            def _():
                sum_sc[...] += s_blk
                sq_sc[...] += q_blk

        if l == 0:
            deg_out[...] = deg

        @pl.when(p == 1)
        def _():
            t = t_sc[pl.ds(j * RB, RB), :]
            mu = sum_sc[...] / N
            var = sq_sc[...] / N - mu * mu
            hn = jnp.maximum(
                (t - mu) / jnp.sqrt(var + EPS) * g_ref[...] + b_ref[...], 0.0)
            if split_out:
                hout_ref[0] = hn[:, :HH]
                hout_ref[1] = hn[:, HH:]
            else:
                hout_ref[...] = hn

    def _pj(j_of):
        return lambda p, j: j_of(jnp.where(p == 0, j, NRB - 1))

    if l == 0:
        dh_specs = [pl.BlockSpec((1, RB, HH), _pj(lambda j: (0, j, 0))),
                    pl.BlockSpec((1, RB, HH), _pj(lambda j: (1, j, 0)))]
    else:
        dh_specs = [pl.BlockSpec((RB, 1), _pj(lambda j: (j, 0)))]
    in_specs = [
        pl.BlockSpec((1, RB, HH), _pj(lambda j: (0, j, 0))),
        pl.BlockSpec((1, RB, HH), _pj(lambda j: (1, j, 0))),
        pl.BlockSpec((1, RB, HH), _pj(lambda j: (0, j, 0))),
        pl.BlockSpec((1, RB, HH), _pj(lambda j: (1, j, 0))),
    ] + dh_specs + [
        pl.BlockSpec((H, H), lambda p, j: (0, 0)),
        pl.BlockSpec((1, H), lambda p, j: (0, 0)),
        pl.BlockSpec((H, H), lambda p, j: (0, 0)),
        pl.BlockSpec((1, H), lambda p, j: (0, 0)),
        pl.BlockSpec((1, H), lambda p, j: (0, 0)),
    ]
    if split_out:
        out_specs = [pl.BlockSpec((2, RB, HH), lambda p, j: (0, j, 0))]
        out_shape = [jax.ShapeDtypeStruct((2, N, HH), _f32)]
    else:
        out_specs = [pl.BlockSpec((RB, H), lambda p, j: (j, 0))]
        out_shape = [jax.ShapeDtypeStruct((N, H), _f32)]
    if l == 0:
        out_specs.append(pl.BlockSpec(
            (RB, 1), lambda p, j: (jnp.where(p == 0, j, NRB - 1), 0)))
        out_shape.append(jax.ShapeDtypeStruct((N, 1), _f32))
    return pl.pallas_call(
        body,
        grid=(2, NRB),
        in_specs=in_specs,
        out_specs=out_specs,
        out_shape=out_shape,
        scratch_shapes=[pltpu.VMEM((N, H), _f32),
                        pltpu.VMEM((1, H), _f32),
                        pltpu.VMEM((1, H), _f32)],
    )


_layer0 = _make_layer(0)
_layer1 = _make_layer(1)
_layer2 = _make_layer(2)


def _heads_body(h_ref, We_ref, be_ref, Wo1_ref, bo1_ref, Wo2_ref, bo2_ref,
                Wd1_ref, bd1_ref, Wd2_ref, bd2_ref,
                emb_ref, obj_ref, dev_ref, sum_sc, max_sc):
    i = pl.program_id(0)
    h = h_ref[...]
    emb_ref[...] = _dotT(h, We_ref[...]) + be_ref[...]
    t1 = jnp.maximum(_dotT(h, Wo1_ref[...]) + bo1_ref[...], 0.0)
    obj_ref[...] = (_dotT(t1, Wo2_ref[...]) + bo2_ref[...])[:, :NOBJ]
    s_blk = jnp.sum(h, axis=0, keepdims=True)
    m_blk = jnp.max(h, axis=0, keepdims=True)

    @pl.when(i == 0)
    def _():
        sum_sc[...] = s_blk
        max_sc[...] = m_blk

    @pl.when(i > 0)
    def _():
        sum_sc[...] += s_blk
        max_sc[...] = jnp.maximum(max_sc[...], m_blk)

    grepr = jnp.concatenate([sum_sc[...] / N, max_sc[...]], axis=1)
    d1 = jnp.maximum(_dotT(grepr, Wd1_ref[...]) + bd1_ref[...], 0.0)
    dev_ref[...] = (_dotT(d1, Wd2_ref[...]) + bd2_ref[...])[:, :NDEV]


_heads_call = pl.pallas_call(
    _heads_body,
    grid=(NRB,),
    in_specs=[
        pl.BlockSpec((RB, H), lambda i: (i, 0)),
        pl.BlockSpec((H, H), lambda i: (0, 0)),
        pl.BlockSpec((1, H), lambda i: (0, 0)),
        pl.BlockSpec((H, H), lambda i: (0, 0)),
        pl.BlockSpec((1, H), lambda i: (0, 0)),
        pl.BlockSpec((NOBJ_PAD, H), lambda i: (0, 0)),
        pl.BlockSpec((1, NOBJ_PAD), lambda i: (0, 0)),
        pl.BlockSpec((H, 2 * H), lambda i: (0, 0)),
        pl.BlockSpec((1, H), lambda i: (0, 0)),
        pl.BlockSpec((NDEV_PAD, H), lambda i: (0, 0)),
        pl.BlockSpec((1, NDEV_PAD), lambda i: (0, 0)),
    ],
    out_specs=[
        pl.BlockSpec((RB, H), lambda i: (i, 0)),
        pl.BlockSpec((RB, NOBJ), lambda i: (i, 0)),
        pl.BlockSpec((1, NDEV), lambda i: (0, 0)),
    ],
    out_shape=[jax.ShapeDtypeStruct((N, H), _f32),
               jax.ShapeDtypeStruct((N, NOBJ), _f32),
               jax.ShapeDtypeStruct((1, NDEV), _f32)],
    scratch_shapes=[pltpu.VMEM((1, H), _f32),
                    pltpu.VMEM((1, H), _f32)],
)


def kernel(x, edge_index, edge_attr,
           W_in, b_in, bn_in_g, bn_in_b,
           W_emb, b_emb,
           W_o1, b_o1, W_o2, b_o2,
           W_d1, b_d1, W_d2, b_d2,
           sage0_Wl, sage0_bl, sage0_Wr, norm0_g, norm0_b,
           sage1_Wl, sage1_bl, sage1_Wr, norm1_g, norm1_b,
           sage2_Wl, sage2_bl, sage2_Wr, norm2_g, norm2_b):
    row = lambda v: v.reshape(1, -1)
    ei4 = edge_index.reshape(2, NSUB * NIB, CPB, K)
    zacc = jnp.zeros((RPS, HH), _f32)
    onesI = jnp.ones((K, HH), _f32)

    (h0,) = _input_call(x, W_in, row(b_in), row(bn_in_g), row(bn_in_b))
    deg128 = _deg_call(ei4, zacc, onesI)
    a0 = _agg(h0.reshape(2 * N, HH), ei4, zacc)
    h1, deg = _layer0(a0, a0, h0, h0, deg128, deg128,
                      sage0_Wl, row(sage0_bl), sage0_Wr,
                      row(norm0_g), row(norm0_b))
    a1 = _agg(h1.reshape(2 * N, HH), ei4, zacc)
    (h2,) = _layer1(a1, a1, h1, h1, deg,
                    sage1_Wl, row(sage1_bl), sage1_Wr,
                    row(norm1_g), row(norm1_b))
    a2 = _agg(h2.reshape(2 * N, HH), ei4, zacc)
    (h3,) = _layer2(a2, a2, h2, h2, deg,
                    sage2_Wl, row(sage2_bl), sage2_Wr,
                    row(norm2_g), norm2_b.reshape(1, -1))

    W_o2p = jnp.pad(W_o2, ((0, NOBJ_PAD - NOBJ), (0, 0)))
    b_o2p = jnp.pad(b_o2, (0, NOBJ_PAD - NOBJ)).reshape(1, -1)
    W_d2p = jnp.pad(W_d2, ((0, NDEV_PAD - NDEV), (0, 0)))
    b_d2p = jnp.pad(b_d2, (0, NDEV_PAD - NDEV)).reshape(1, -1)
    emb, obj, dev = _heads_call(h3, W_emb, row(b_emb),
                                W_o1, row(b_o1), W_o2p, b_o2p,
                                W_d1, row(b_d1), W_d2p, b_d2p)
    return emb, obj, dev, h3
